# baseline (speedup 1.0000x reference)
import functools

import jax
import jax.numpy as jnp
from jax import lax
from jax.experimental import pallas as pl
from jax.experimental.pallas import tpu as pltpu
from jax.experimental.pallas import tpu_sc as plsc

_B = 16384
_D = 32
_PACK = 4
_PD = _D * _PACK
_NC = 2
_NS = 16
_NW = _NC * _NS
_BPW = _B // _NW


def _gather_body(user_tab, item_tab, uidx, iidx, uout, iout,
                 uidx_v, iidx_v, rows_v, sem):
    wid = lax.axis_index("s") * _NC + lax.axis_index("c")
    base = wid * _BPW
    pltpu.sync_copy(uidx.at[pl.ds(base, _BPW)], uidx_v)
    pltpu.sync_copy(iidx.at[pl.ds(base, _BPW)], iidx_v)
    pltpu.async_copy(user_tab.at[uidx_v], rows_v, sem).wait()
    pltpu.sync_copy(rows_v, uout.at[pl.ds(base, _BPW)])
    pltpu.async_copy(item_tab.at[iidx_v], rows_v, sem).wait()
    pltpu.sync_copy(rows_v, iout.at[pl.ds(base, _BPW)])


@functools.lru_cache(maxsize=None)
def _build_gather2():
    mesh = plsc.VectorSubcoreMesh(
        core_axis_name="c", subcore_axis_name="s",
        num_cores=_NC, num_subcores=_NS,
    )
    return pl.kernel(
        _gather_body,
        out_type=(
            jax.ShapeDtypeStruct((_B, _PD), jnp.float32),
            jax.ShapeDtypeStruct((_B, _PD), jnp.float32),
        ),
        mesh=mesh,
        scratch_types=[
            pltpu.VMEM((_BPW,), jnp.int32),
            pltpu.VMEM((_BPW,), jnp.int32),
            pltpu.VMEM((_BPW, _PD), jnp.float32),
            pltpu.SemaphoreType.DMA,
        ],
    )


_BS = 2048


def _mlp_body(desc_ref, uraw_ref, iraw_ref, uoff_ref, ioff_ref,
              wd_ref, bd_ref, w1u_ref, w1i_ref, w1d_ref, b1_ref,
              w2_ref, b2_ref, wo_ref, bo_ref, out_ref):
    uraw = uraw_ref[...]
    iraw = iraw_ref[...]
    uoff = uoff_ref[...]
    ioff = ioff_ref[...]
    u = jnp.zeros((_BS, _D), jnp.float32)
    it = jnp.zeros((_BS, _D), jnp.float32)
    for k in range(_PACK):
        umask = (uoff == k).astype(jnp.float32)
        imask = (ioff == k).astype(jnp.float32)
        u = u + umask * uraw[:, k * _D:(k + 1) * _D]
        it = it + imask * iraw[:, k * _D:(k + 1) * _D]
    d = jnp.dot(desc_ref[...], wd_ref[...], preferred_element_type=jnp.float32)
    d = jnp.maximum(d + bd_ref[...], 0.0)
    h1 = jnp.dot(u, w1u_ref[...], preferred_element_type=jnp.float32)
    h1 = h1 + jnp.dot(it, w1i_ref[...], preferred_element_type=jnp.float32)
    h1 = h1 + jnp.dot(d, w1d_ref[...], preferred_element_type=jnp.float32)
    h1 = jnp.maximum(h1 + b1_ref[...], 0.0)
    h2 = jnp.dot(h1, w2_ref[...], preferred_element_type=jnp.float32)
    h2 = jnp.maximum(h2 + b2_ref[...], 0.0)
    out_ref[...] = jnp.dot(h2, wo_ref[...], preferred_element_type=jnp.float32) + bo_ref[...]


def _mlp(desc, u_raw, i_raw, uoff, ioff, wd, bd, w1u, w1i, w1d, b1, w2, b2, wo, bo):
    grid = (_B // _BS,)
    full = lambda shape: pl.BlockSpec(shape, lambda i: (0, 0))
    return pl.pallas_call(
        _mlp_body,
        grid=grid,
        in_specs=[
            pl.BlockSpec((_BS, 300), lambda i: (i, 0)),
            pl.BlockSpec((_BS, _PD), lambda i: (i, 0)),
            pl.BlockSpec((_BS, _PD), lambda i: (i, 0)),
            pl.BlockSpec((_BS, 1), lambda i: (i, 0)),
            pl.BlockSpec((_BS, 1), lambda i: (i, 0)),
            full((300, _D)),
            full((1, _D)),
            full((_D, 64)),
            full((_D, 64)),
            full((_D, 64)),
            full((1, 64)),
            full((64, 32)),
            full((1, 32)),
            full((32, 1)),
            full((1, 1)),
        ],
        out_specs=pl.BlockSpec((_BS, 1), lambda i: (i, 0)),
        out_shape=jax.ShapeDtypeStruct((_B, 1), jnp.float32),
    )(desc, u_raw, i_raw, uoff, ioff, wd, bd, w1u, w1i, w1d, b1, w2, b2, wo, bo)


def kernel(user_input, item_input, description_input, user_table, item_table,
           W_desc, b_desc, W1, b1, W2, b2, W_out, b_out):
    utab4 = user_table.reshape(-1, _PD)
    itab4 = item_table.reshape(-1, _PD)
    uidx = user_input.reshape(-1)
    iidx = item_input.reshape(-1)
    u_raw, i_raw = _build_gather2()(utab4, itab4, uidx // _PACK, iidx // _PACK)
    uoff = (user_input % _PACK).astype(jnp.int32)
    ioff = (item_input % _PACK).astype(jnp.int32)
    return _mlp(
        description_input, u_raw, i_raw, uoff, ioff,
        W_desc, b_desc.reshape(1, -1),
        W1[:_D], W1[_D:2 * _D], W1[2 * _D:], b1.reshape(1, -1),
        W2, b2.reshape(1, -1),
        W_out, b_out.reshape(1, -1),
    )


# speedup vs baseline: 1.0345x; 1.0345x over previous
import functools

import jax
import jax.numpy as jnp
from jax import lax
from jax.experimental import pallas as pl
from jax.experimental.pallas import tpu as pltpu
from jax.experimental.pallas import tpu_sc as plsc

_B = 16384
_D = 32
_PACK = 4
_PD = _D * _PACK
_NC = 2
_NS = 16
_NW = _NC * _NS
_BPW = _B // _NW


def _gather_body(user_tab, item_tab, uidx, iidx, uout, iout,
                 uidx_v, iidx_v, rows_v, sem):
    wid = lax.axis_index("s") * _NC + lax.axis_index("c")
    base = wid * _BPW
    pltpu.sync_copy(uidx.at[pl.ds(base, _BPW)], uidx_v)
    pltpu.sync_copy(iidx.at[pl.ds(base, _BPW)], iidx_v)
    pltpu.async_copy(user_tab.at[uidx_v], rows_v, sem).wait()
    pltpu.sync_copy(rows_v, uout.at[pl.ds(base, _BPW)])
    pltpu.async_copy(item_tab.at[iidx_v], rows_v, sem).wait()
    pltpu.sync_copy(rows_v, iout.at[pl.ds(base, _BPW)])


@functools.lru_cache(maxsize=None)
def _build_gather2():
    mesh = plsc.VectorSubcoreMesh(
        core_axis_name="c", subcore_axis_name="s",
        num_cores=_NC, num_subcores=_NS,
    )
    return pl.kernel(
        _gather_body,
        out_type=(
            jax.ShapeDtypeStruct((_B, _PD), jnp.float32),
            jax.ShapeDtypeStruct((_B, _PD), jnp.float32),
        ),
        mesh=mesh,
        scratch_types=[
            pltpu.VMEM((_BPW,), jnp.int32),
            pltpu.VMEM((_BPW,), jnp.int32),
            pltpu.VMEM((_BPW, _PD), jnp.float32),
            pltpu.SemaphoreType.DMA,
        ],
    )


_BS = 2048
_NB = _B // _BS


def _mlp_body(descT_ref, uraw_ref, iraw_ref, uoff_ref, ioff_ref,
              wdT_ref, bdT_ref, w1uT_ref, w1iT_ref, w1dT_ref, b1T_ref,
              w2T_ref, b2T_ref, woT_ref, bo_ref, out_ref):
    f32 = jnp.float32
    uraw = uraw_ref[...]
    iraw = iraw_ref[...]
    uoff = uoff_ref[...]
    ioff = ioff_ref[...]
    u = jnp.zeros((_BS, _D), f32)
    it = jnp.zeros((_BS, _D), f32)
    for k in range(_PACK):
        umask = (uoff == k).astype(f32)
        imask = (ioff == k).astype(f32)
        u = u + umask * uraw[:, k * _D:(k + 1) * _D]
        it = it + imask * iraw[:, k * _D:(k + 1) * _D]
    dT = lax.dot_general(wdT_ref[...], descT_ref[...], (((1,), (0,)), ((), ())),
                         preferred_element_type=f32)
    dT = jnp.maximum(dT + bdT_ref[...], 0.0)
    h1T = lax.dot_general(w1uT_ref[...], u, (((1,), (1,)), ((), ())),
                          preferred_element_type=f32)
    h1T = h1T + lax.dot_general(w1iT_ref[...], it, (((1,), (1,)), ((), ())),
                                preferred_element_type=f32)
    h1T = h1T + lax.dot_general(w1dT_ref[...], dT, (((1,), (0,)), ((), ())),
                                preferred_element_type=f32)
    h1T = jnp.maximum(h1T + b1T_ref[...], 0.0)
    h2T = lax.dot_general(w2T_ref[...], h1T, (((1,), (0,)), ((), ())),
                          preferred_element_type=f32)
    h2T = jnp.maximum(h2T + b2T_ref[...], 0.0)
    outT = lax.dot_general(woT_ref[...], h2T, (((1,), (0,)), ((), ())),
                           preferred_element_type=f32)
    out_ref[...] = (outT + bo_ref[...]).reshape(1, 1, _BS)


def _mlp(descT, u_raw, i_raw, uoff, ioff, wdT, bdT, w1uT, w1iT, w1dT, b1T,
         w2T, b2T, woT, bo):
    full = lambda shape: pl.BlockSpec(shape, lambda i: tuple(0 for _ in shape))
    return pl.pallas_call(
        _mlp_body,
        grid=(_NB,),
        in_specs=[
            pl.BlockSpec((300, _BS), lambda i: (0, i)),
            pl.BlockSpec((_BS, _PD), lambda i: (i, 0)),
            pl.BlockSpec((_BS, _PD), lambda i: (i, 0)),
            pl.BlockSpec((_BS, 1), lambda i: (i, 0)),
            pl.BlockSpec((_BS, 1), lambda i: (i, 0)),
            full((_D, 300)),
            full((_D, 1)),
            full((64, _D)),
            full((64, _D)),
            full((64, _D)),
            full((64, 1)),
            full((_D, 64)),
            full((_D, 1)),
            full((1, _D)),
            full((1, 1)),
        ],
        out_specs=pl.BlockSpec((1, 1, _BS), lambda i: (i, 0, 0)),
        out_shape=jax.ShapeDtypeStruct((_NB, 1, _BS), jnp.float32),
    )(descT, u_raw, i_raw, uoff, ioff, wdT, bdT, w1uT, w1iT, w1dT, b1T,
      w2T, b2T, woT, bo)


def kernel(user_input, item_input, description_input, user_table, item_table,
           W_desc, b_desc, W1, b1, W2, b2, W_out, b_out):
    utab4 = user_table.reshape(-1, _PD)
    itab4 = item_table.reshape(-1, _PD)
    uidx = user_input.reshape(-1)
    iidx = item_input.reshape(-1)
    u_raw, i_raw = _build_gather2()(utab4, itab4, uidx // _PACK, iidx // _PACK)
    uoff = (user_input % _PACK).astype(jnp.int32)
    ioff = (item_input % _PACK).astype(jnp.int32)
    W1T = W1.T
    out3 = _mlp(
        description_input.T, u_raw, i_raw, uoff, ioff,
        W_desc.T, b_desc.reshape(-1, 1),
        W1T[:, :_D], W1T[:, _D:2 * _D], W1T[:, 2 * _D:], b1.reshape(-1, 1),
        W2.T, b2.reshape(-1, 1),
        W_out.T, b_out.reshape(1, 1),
    )
    return out3.reshape(_B, 1)


# speedup vs baseline: 1.0769x; 1.0410x over previous
import functools

import jax
import jax.numpy as jnp
from jax import lax
from jax.experimental import pallas as pl
from jax.experimental.pallas import tpu as pltpu
from jax.experimental.pallas import tpu_sc as plsc

_B = 16384
_D = 32
_NC = 2
_NS = 16
_NW = _NC * _NS
_BPW = _B // _NW


def _gather_body(user_tab, item_tab, uidx, iidx, uout, iout,
                 uidx_v, iidx_v, urows_v, irows_v, sem_u, sem_i):
    wid = lax.axis_index("s") * _NC + lax.axis_index("c")
    base = wid * _BPW
    pltpu.sync_copy(uidx.at[pl.ds(base, _BPW)], uidx_v)
    pltpu.sync_copy(iidx.at[pl.ds(base, _BPW)], iidx_v)
    cu = pltpu.async_copy(user_tab.at[uidx_v], urows_v, sem_u)
    ci = pltpu.async_copy(item_tab.at[iidx_v], irows_v, sem_i)
    cu.wait()
    ci.wait()
    pltpu.sync_copy(urows_v, uout.at[pl.ds(base, _BPW)])
    pltpu.sync_copy(irows_v, iout.at[pl.ds(base, _BPW)])


@functools.lru_cache(maxsize=None)
def _build_gather2():
    mesh = plsc.VectorSubcoreMesh(
        core_axis_name="c", subcore_axis_name="s",
        num_cores=_NC, num_subcores=_NS,
    )
    return pl.kernel(
        _gather_body,
        out_type=(
            jax.ShapeDtypeStruct((_B, _D), jnp.float32),
            jax.ShapeDtypeStruct((_B, _D), jnp.float32),
        ),
        mesh=mesh,
        compiler_params=pltpu.CompilerParams(use_tc_tiling_on_sc=False),
        scratch_types=[
            pltpu.VMEM((_BPW,), jnp.int32),
            pltpu.VMEM((_BPW,), jnp.int32),
            pltpu.VMEM((_BPW, _D), jnp.float32),
            pltpu.VMEM((_BPW, _D), jnp.float32),
            pltpu.SemaphoreType.DMA,
            pltpu.SemaphoreType.DMA,
        ],
    )


_BS = 2048
_NB = _B // _BS


def _mlp_body(descT_ref, u_ref, i_ref,
              wdT_ref, bdT_ref, w1uT_ref, w1iT_ref, w1dT_ref, b1T_ref,
              w2T_ref, b2T_ref, woT_ref, bo_ref, out_ref):
    f32 = jnp.float32
    dT = lax.dot_general(wdT_ref[...], descT_ref[...], (((1,), (0,)), ((), ())),
                         preferred_element_type=f32)
    dT = jnp.maximum(dT + bdT_ref[...], 0.0)
    h1T = lax.dot_general(w1uT_ref[...], u_ref[...], (((1,), (1,)), ((), ())),
                          preferred_element_type=f32)
    h1T = h1T + lax.dot_general(w1iT_ref[...], i_ref[...], (((1,), (1,)), ((), ())),
                                preferred_element_type=f32)
    h1T = h1T + lax.dot_general(w1dT_ref[...], dT, (((1,), (0,)), ((), ())),
                                preferred_element_type=f32)
    h1T = jnp.maximum(h1T + b1T_ref[...], 0.0)
    h2T = lax.dot_general(w2T_ref[...], h1T, (((1,), (0,)), ((), ())),
                          preferred_element_type=f32)
    h2T = jnp.maximum(h2T + b2T_ref[...], 0.0)
    outT = lax.dot_general(woT_ref[...], h2T, (((1,), (0,)), ((), ())),
                           preferred_element_type=f32)
    out_ref[...] = (outT + bo_ref[...]).reshape(1, 1, _BS)


def _mlp(descT, u_emb, i_emb, wdT, bdT, w1uT, w1iT, w1dT, b1T,
         w2T, b2T, woT, bo):
    full = lambda shape: pl.BlockSpec(shape, lambda i: tuple(0 for _ in shape))
    return pl.pallas_call(
        _mlp_body,
        grid=(_NB,),
        in_specs=[
            pl.BlockSpec((300, _BS), lambda i: (0, i)),
            pl.BlockSpec((_BS, _D), lambda i: (i, 0)),
            pl.BlockSpec((_BS, _D), lambda i: (i, 0)),
            full((_D, 300)),
            full((_D, 1)),
            full((64, _D)),
            full((64, _D)),
            full((64, _D)),
            full((64, 1)),
            full((_D, 64)),
            full((_D, 1)),
            full((1, _D)),
            full((1, 1)),
        ],
        out_specs=pl.BlockSpec((1, 1, _BS), lambda i: (i, 0, 0)),
        out_shape=jax.ShapeDtypeStruct((_NB, 1, _BS), jnp.float32),
    )(descT, u_emb, i_emb, wdT, bdT, w1uT, w1iT, w1dT, b1T,
      w2T, b2T, woT, bo)


def kernel(user_input, item_input, description_input, user_table, item_table,
           W_desc, b_desc, W1, b1, W2, b2, W_out, b_out):
    uidx = user_input.reshape(-1)
    iidx = item_input.reshape(-1)
    u_emb, i_emb = _build_gather2()(user_table, item_table, uidx, iidx)
    W1T = W1.T
    out3 = _mlp(
        description_input.T, u_emb, i_emb,
        W_desc.T, b_desc.reshape(-1, 1),
        W1T[:, :_D], W1T[:, _D:2 * _D], W1T[:, 2 * _D:], b1.reshape(-1, 1),
        W2.T, b2.reshape(-1, 1),
        W_out.T, b_out.reshape(1, 1),
    )
    return out3.reshape(_B, 1)
